# baseline (speedup 1.0000x reference)
import functools

import jax
import jax.numpy as jnp
from jax import lax
from jax.experimental import pallas as pl
from jax.experimental.pallas import tpu as pltpu
from jax.experimental.pallas import tpu_sc as plsc

NUM_FEAT = 26
VOCAB = 100000
EMBED_DIM = 32
BATCH = 16384

NUM_CORES = 2
NUM_SUBCORES = 16
NUM_WORKERS = NUM_CORES * NUM_SUBCORES
BPW = BATCH // NUM_WORKERS


def _emb_body(*refs):
    feats = refs[:NUM_FEAT]
    tables = refs[NUM_FEAT:2 * NUM_FEAT]
    out = refs[2 * NUM_FEAT]
    idx_v, rows_v, sem = refs[2 * NUM_FEAT + 1:]

    wid = lax.axis_index("s") * NUM_CORES + lax.axis_index("c")
    base = wid * BPW
    for i in range(NUM_FEAT):
        pltpu.sync_copy(feats[i].at[pl.ds(base, BPW)], idx_v)
        pltpu.async_copy(tables[i].at[idx_v], rows_v, sem).wait()
        pltpu.sync_copy(rows_v, out.at[pl.ds(base, BPW), i])


@functools.partial(jax.jit, static_argnums=())
def kernel(*args):
    mesh = plsc.VectorSubcoreMesh(
        core_axis_name="c", subcore_axis_name="s",
        num_cores=NUM_CORES, num_subcores=NUM_SUBCORES,
    )
    out3 = pl.kernel(
        _emb_body,
        out_type=jax.ShapeDtypeStruct((BATCH, NUM_FEAT, EMBED_DIM), jnp.float32),
        mesh=mesh,
        scratch_types=[
            pltpu.VMEM((BPW,), jnp.int32),
            pltpu.VMEM((BPW, EMBED_DIM), jnp.float32),
            pltpu.SemaphoreType.DMA,
        ],
        compiler_params=pltpu.CompilerParams(use_tc_tiling_on_sc=False),
    )(*args)
    return out3.reshape(BATCH, NUM_FEAT * EMBED_DIM)


# speedup vs baseline: 1.0225x; 1.0225x over previous
import functools

import jax
import jax.numpy as jnp
from jax import lax
from jax.experimental import pallas as pl
from jax.experimental.pallas import tpu as pltpu
from jax.experimental.pallas import tpu_sc as plsc

NUM_FEAT = 26
VOCAB = 100000
EMBED_DIM = 32
BATCH = 16384

NUM_CORES = 2
NUM_SUBCORES = 16
NUM_WORKERS = NUM_CORES * NUM_SUBCORES
BPW = BATCH // NUM_WORKERS

NBUF = 4


def _emb_body(*refs):
    feats = refs[:NUM_FEAT]
    tables = refs[NUM_FEAT:2 * NUM_FEAT]
    out = refs[2 * NUM_FEAT]
    idx_v = refs[2 * NUM_FEAT + 1]
    rows_v = refs[2 * NUM_FEAT + 2]
    isem = refs[2 * NUM_FEAT + 3]
    gsems = refs[2 * NUM_FEAT + 4:2 * NUM_FEAT + 4 + NBUF]
    wsems = refs[2 * NUM_FEAT + 4 + NBUF:]

    wid = lax.axis_index("s") * NUM_CORES + lax.axis_index("c")
    base = wid * BPW

    idx_descs = [
        pltpu.async_copy(feats[i].at[pl.ds(base, BPW)], idx_v.at[i], isem)
        for i in range(NUM_FEAT)
    ]
    for d in idx_descs:
        d.wait()

    def gather(i):
        b = i % NBUF
        return pltpu.async_copy(tables[i].at[idx_v.at[i]], rows_v.at[b],
                                gsems[b])

    def writeback(i):
        b = i % NBUF
        return pltpu.async_copy(rows_v.at[b], out.at[pl.ds(base, BPW), i],
                                wsems[b])

    g_descs = [None] * NUM_FEAT
    w_descs = [None] * NUM_FEAT
    for i in range(min(NBUF, NUM_FEAT)):
        g_descs[i] = gather(i)
    for i in range(NUM_FEAT):
        g_descs[i].wait()
        w_descs[i] = writeback(i)
        if i + NBUF < NUM_FEAT:
            w_descs[i].wait()
            g_descs[i + NBUF] = gather(i + NBUF)
    for i in range(max(0, NUM_FEAT - NBUF), NUM_FEAT):
        w_descs[i].wait()


@functools.partial(jax.jit, static_argnums=())
def kernel(*args):
    mesh = plsc.VectorSubcoreMesh(
        core_axis_name="c", subcore_axis_name="s",
        num_cores=NUM_CORES, num_subcores=NUM_SUBCORES,
    )
    out3 = pl.kernel(
        _emb_body,
        out_type=jax.ShapeDtypeStruct((BATCH, NUM_FEAT, EMBED_DIM), jnp.float32),
        mesh=mesh,
        scratch_types=(
            [pltpu.VMEM((NUM_FEAT, BPW), jnp.int32),
             pltpu.VMEM((NBUF, BPW, EMBED_DIM), jnp.float32),
             pltpu.SemaphoreType.DMA]
            + [pltpu.SemaphoreType.DMA] * NBUF
            + [pltpu.SemaphoreType.DMA] * NBUF
        ),
        compiler_params=pltpu.CompilerParams(use_tc_tiling_on_sc=False),
    )(*args)
    return out3.reshape(BATCH, NUM_FEAT * EMBED_DIM)
